# baseline (speedup 1.0000x reference)
import functools

import jax
import jax.numpy as jnp
from jax import lax
from jax.experimental import pallas as pl
from jax.experimental.pallas import tpu as pltpu
from jax.experimental.pallas import tpu_sc as plsc

_B = 16384
_S = 200
_D = 64
_IDXROW = 128
_LANE_TILE = 128
_SUB = 8
_S_STRIDE = _D * _B
_DT_STRIDE = _SUB * _LANE_TILE * _LANE_TILE


def _raw1d(seq):
    x = jnp.transpose(seq, (1, 2, 0))
    x = x.reshape(_S, _D // _SUB, _SUB, _B // _LANE_TILE, _LANE_TILE)
    x = jnp.transpose(x, (0, 1, 3, 2, 4))
    return x.reshape(-1)


def _make_sc_gather():
    info = plsc.get_sparse_core_info()
    nc, ns = info.num_cores, info.num_subcores
    nw = nc * ns
    b_per_w = _B // nw
    n_chunks = b_per_w // _IDXROW

    mesh = plsc.VectorSubcoreMesh(core_axis_name="c", subcore_axis_name="s")

    @functools.partial(
        pl.kernel,
        mesh=mesh,
        out_type=jax.ShapeDtypeStruct((_D, _B), jnp.float32),
        scratch_types=[
            pltpu.VMEM((n_chunks, _IDXROW), jnp.int32),
            pltpu.VMEM((n_chunks, _IDXROW), jnp.int32),
            pltpu.VMEM((_D, n_chunks, _IDXROW), jnp.int32),
            pltpu.VMEM((_D, b_per_w), jnp.float32),
            pltpu.SemaphoreType.DMA,
        ],
    )
    def gather_kernel(raw_hbm, idx_hbm, out_hbm, idx_v, sbig_v, idx_vm,
                      out_v, sem):
        wid = lax.axis_index("s") * nc + lax.axis_index("c")
        base = wid * b_per_w
        bt_base = base // _LANE_TILE

        pltpu.sync_copy(idx_hbm.at[pl.ds(wid * n_chunks, n_chunks)], idx_v)

        lane = lax.iota(jnp.int32, 16)
        for c in range(n_chunks):
            colbase = (bt_base + c) * (_SUB * _LANE_TILE)
            for g in range(_IDXROW // 16):
                s_vec = idx_v[c, pl.ds(g * 16, 16)]
                sbig_v[c, pl.ds(g * 16, 16)] = (
                    s_vec * _S_STRIDE + (lane + (colbase + g * 16))
                )

        def fill_body(d):
            dconst = (d // _SUB) * _DT_STRIDE + (d % _SUB) * _LANE_TILE
            for c in range(n_chunks):
                for g in range(_IDXROW // 16):
                    v = sbig_v[c, pl.ds(g * 16, 16)]
                    idx_vm[d, c, pl.ds(g * 16, 16)] = v + dconst

        K = 16
        PRE = 18

        def fire(d):
            for c in range(n_chunks):
                pltpu.async_copy(
                    raw_hbm.at[idx_vm.at[d, c]],
                    out_v.at[d, pl.ds(c * _IDXROW, _IDXROW)],
                    sem,
                )

        def drain_row():
            pltpu.make_async_copy(
                raw_hbm.at[pl.ds(0, b_per_w)], out_v.at[0], sem
            ).wait()

        pl.loop(0, PRE)(fill_body)
        for d in range(K):
            fire(d)

        def steady_body(d):
            fire(d)
            fill_body(d + (PRE - K))
            drain_row()

        pl.loop(K, _D - (PRE - K), unroll=2)(steady_body)

        def tail_body(d):
            fire(d)
            drain_row()

        pl.loop(_D - (PRE - K), _D)(tail_body)
        for _ in range(K):
            drain_row()

        pltpu.sync_copy(out_v, out_hbm.at[:, pl.ds(base, b_per_w)])

    return gather_kernel


_sc_gather = _make_sc_gather()


@jax.jit
def kernel(seq, idx):
    raw = _raw1d(seq)
    idx2d = idx.reshape(-1).astype(jnp.int32).reshape(_B // _IDXROW, _IDXROW)
    out_t = _sc_gather(raw, idx2d)
    return jnp.transpose(out_t, (1, 0))


# speedup vs baseline: 1.0387x; 1.0387x over previous
import functools

import jax
import jax.numpy as jnp
from jax import lax
from jax.experimental import pallas as pl
from jax.experimental.pallas import tpu as pltpu
from jax.experimental.pallas import tpu_sc as plsc

_B = 16384
_S = 200
_D = 64
_IDXROW = 128
_LANE_TILE = 128
_SUB = 8
_S_STRIDE = _D * _B
_DT_STRIDE = _SUB * _LANE_TILE * _LANE_TILE


def _raw1d(seq):
    x = jnp.transpose(seq, (1, 2, 0))
    x = x.reshape(_S, _D // _SUB, _SUB, _B // _LANE_TILE, _LANE_TILE)
    x = jnp.transpose(x, (0, 1, 3, 2, 4))
    return x.reshape(-1)


def _make_sc_gather():
    info = plsc.get_sparse_core_info()
    nc, ns = info.num_cores, info.num_subcores
    nw = nc * ns
    b_per_w = _B // nw
    n_chunks = b_per_w // _IDXROW

    mesh = plsc.VectorSubcoreMesh(core_axis_name="c", subcore_axis_name="s")

    @functools.partial(
        pl.kernel,
        mesh=mesh,
        out_type=jax.ShapeDtypeStruct((_D, _B), jnp.float32),
        scratch_types=[
            pltpu.VMEM((n_chunks, _IDXROW), jnp.int32),
            pltpu.VMEM((n_chunks, _IDXROW), jnp.int32),
            pltpu.VMEM((_D, n_chunks, _IDXROW), jnp.int32),
            pltpu.VMEM((_D, b_per_w), jnp.float32),
            pltpu.SemaphoreType.DMA,
        ],
    )
    def gather_kernel(raw_hbm, idx_hbm, out_hbm, idx_v, sbig_v, idx_vm,
                      out_v, sem):
        wid = lax.axis_index("s") * nc + lax.axis_index("c")
        base = wid * b_per_w
        bt_base = base // _LANE_TILE

        pltpu.sync_copy(idx_hbm.at[pl.ds(wid * n_chunks, n_chunks)], idx_v)

        lane = lax.iota(jnp.int32, 16)
        for c in range(n_chunks):
            colbase = (bt_base + c) * (_SUB * _LANE_TILE)
            for g in range(_IDXROW // 16):
                s_vec = idx_v[c, pl.ds(g * 16, 16)]
                sbig_v[c, pl.ds(g * 16, 16)] = (
                    s_vec * _S_STRIDE + (lane + (colbase + g * 16))
                )

        def fill_body(d):
            dconst = (d // _SUB) * _DT_STRIDE + (d % _SUB) * _LANE_TILE
            for c in range(n_chunks):
                for g in range(_IDXROW // 16):
                    v = sbig_v[c, pl.ds(g * 16, 16)]
                    idx_vm[d, c, pl.ds(g * 16, 16)] = v + dconst

        K = 24
        PRE = 26

        def fire(d):
            for c in range(n_chunks):
                pltpu.async_copy(
                    raw_hbm.at[idx_vm.at[d, c]],
                    out_v.at[d, pl.ds(c * _IDXROW, _IDXROW)],
                    sem,
                )

        def drain_row():
            pltpu.make_async_copy(
                raw_hbm.at[pl.ds(0, b_per_w)], out_v.at[0], sem
            ).wait()

        pl.loop(0, PRE)(fill_body)
        for d in range(K):
            fire(d)

        def steady_body(d):
            fill_body(d + (PRE - K))
            fire(d)
            drain_row()

        pl.loop(K, _D - (PRE - K))(steady_body)

        def tail_body(d):
            fire(d)
            drain_row()

        pl.loop(_D - (PRE - K), _D)(tail_body)
        for _ in range(K):
            drain_row()

        pltpu.sync_copy(out_v, out_hbm.at[:, pl.ds(base, b_per_w)])

    return gather_kernel


_sc_gather = _make_sc_gather()


@jax.jit
def kernel(seq, idx):
    raw = _raw1d(seq)
    idx2d = idx.reshape(-1).astype(jnp.int32).reshape(_B // _IDXROW, _IDXROW)
    out_t = _sc_gather(raw, idx2d)
    return jnp.transpose(out_t, (1, 0))
